# baseline (speedup 1.0000x reference)
import functools

import jax
import jax.numpy as jnp
from jax import lax
from jax.experimental import pallas as pl
from jax.experimental.pallas import tpu as pltpu
from jax.experimental.pallas import tpu_sc as plsc

NUM_ENTITIES = 50000
EMBED_DIM = 512
HIDDEN_DIM = 1024
BATCH = 16384

_NC = 2
_NS = 16
_NW = _NC * _NS
_BPW = BATCH // _NW
_CH = 64
_NCHUNK = _BPW // _CH


def _sc_gather_body(entity_hbm, rel_hbm, idxs_hbm, idxr_hbm, idxo_hbm,
                    out_s, out_r, out_o,
                    idxs_v, idxr_v, idxo_v, buf0, buf1,
                    gsem, wsem0, wsem1):
    wid = lax.axis_index("s") * _NC + lax.axis_index("c")
    base = wid * _BPW

    pltpu.sync_copy(idxs_hbm.at[pl.ds(base, _BPW)], idxs_v)
    pltpu.sync_copy(idxr_hbm.at[pl.ds(base, _BPW)], idxr_v)
    pltpu.sync_copy(idxo_hbm.at[pl.ds(base, _BPW)], idxo_v)

    jobs = (
        (entity_hbm, idxs_v, out_s),
        (rel_hbm, idxr_v, out_r),
        (entity_hbm, idxo_v, out_o),
    )
    bufs = (buf0, buf1)
    wsems = (wsem0, wsem1)
    pending = [None, None]
    step = 0
    for table, idx_v, out in jobs:
        for k in range(_NCHUNK):
            slot = step % 2
            if pending[slot] is not None:
                pending[slot].wait()
            buf = bufs[slot]
            pltpu.async_copy(
                table.at[idx_v.at[pl.ds(k * _CH, _CH)]], buf, gsem
            ).wait()
            pending[slot] = pltpu.async_copy(
                buf, out.at[pl.ds(base + k * _CH, _CH)], wsems[slot]
            )
            step += 1
    for p in pending:
        if p is not None:
            p.wait()


@functools.cache
def _sc_gather():
    return functools.partial(
        pl.kernel,
        out_type=[jax.ShapeDtypeStruct((BATCH, EMBED_DIM), jnp.float32)] * 3,
        mesh=plsc.VectorSubcoreMesh(core_axis_name="c", subcore_axis_name="s",
                                    num_cores=_NC, num_subcores=_NS),
        scratch_types=[
            pltpu.VMEM((_BPW,), jnp.int32),
            pltpu.VMEM((_BPW,), jnp.int32),
            pltpu.VMEM((_BPW,), jnp.int32),
            pltpu.VMEM((_CH, EMBED_DIM), jnp.float32),
            pltpu.VMEM((_CH, EMBED_DIM), jnp.float32),
            pltpu.SemaphoreType.DMA,
            pltpu.SemaphoreType.DMA,
            pltpu.SemaphoreType.DMA,
        ],
    )(_sc_gather_body)


_BM = 512


def _mlp_body(hs_ref, hr_ref, ho_ref, w1s_ref, w1r_ref, w1o_ref,
              b1_ref, w2t_ref, b2_ref, out_ref):
    bf = jnp.bfloat16
    acc = jnp.dot(hs_ref[...].astype(bf), w1s_ref[...].astype(bf),
                  preferred_element_type=jnp.float32)
    acc += jnp.dot(hr_ref[...].astype(bf), w1r_ref[...].astype(bf),
                   preferred_element_type=jnp.float32)
    acc += jnp.dot(ho_ref[...].astype(bf), w1o_ref[...].astype(bf),
                   preferred_element_type=jnp.float32)
    hidden = jnp.maximum(acc + b1_ref[...], 0.0)
    out_ref[...] = jnp.sum(hidden * w2t_ref[...], axis=1) + b2_ref[0, 0]


def _mlp(hs, hr, ho, w1s, w1r, w1o, b1, w2t, b2):
    grid = (BATCH // _BM,)
    return pl.pallas_call(
        _mlp_body,
        grid=grid,
        in_specs=[
            pl.BlockSpec((_BM, EMBED_DIM), lambda i: (i, 0)),
            pl.BlockSpec((_BM, EMBED_DIM), lambda i: (i, 0)),
            pl.BlockSpec((_BM, EMBED_DIM), lambda i: (i, 0)),
            pl.BlockSpec((EMBED_DIM, HIDDEN_DIM), lambda i: (0, 0)),
            pl.BlockSpec((EMBED_DIM, HIDDEN_DIM), lambda i: (0, 0)),
            pl.BlockSpec((EMBED_DIM, HIDDEN_DIM), lambda i: (0, 0)),
            pl.BlockSpec((1, HIDDEN_DIM), lambda i: (0, 0)),
            pl.BlockSpec((1, HIDDEN_DIM), lambda i: (0, 0)),
            pl.BlockSpec((1, 1), lambda i: (0, 0)),
        ],
        out_specs=pl.BlockSpec((_BM,), lambda i: (i,)),
        out_shape=jax.ShapeDtypeStruct((BATCH,), jnp.float32),
    )(hs, hr, ho, w1s, w1r, w1o, b1, w2t, b2)


def kernel(entity_emb, triples, rel_emb, fc1, fc1_bias, fc2, fc2_bias):
    idx = triples.astype(jnp.int32)
    idx_s = idx[:, 0]
    idx_r = idx[:, 1]
    idx_o = idx[:, 2]
    hs, hr, ho = _sc_gather()(entity_emb, rel_emb, idx_s, idx_r, idx_o)
    w1s = fc1[:EMBED_DIM]
    w1r = fc1[EMBED_DIM:2 * EMBED_DIM]
    w1o = fc1[2 * EMBED_DIM:]
    b1 = fc1_bias.reshape(1, HIDDEN_DIM)
    w2t = fc2.reshape(1, HIDDEN_DIM)
    b2 = fc2_bias.reshape(1, 1)
    return _mlp(hs, hr, ho, w1s, w1r, w1o, b1, w2t, b2)


# speedup vs baseline: 1.0897x; 1.0897x over previous
import functools

import jax
import jax.numpy as jnp
from jax import lax
from jax.experimental import pallas as pl
from jax.experimental.pallas import tpu as pltpu
from jax.experimental.pallas import tpu_sc as plsc

NUM_ENTITIES = 50000
EMBED_DIM = 512
HIDDEN_DIM = 1024
BATCH = 16384

_NC = 2
_NS = 16
_NW = _NC * _NS
_SLABS = 4
_SB = BATCH // _SLABS
_BPW = _SB // _NW
_CH = 64
_NCHUNK = _BPW // _CH


def _sc_gather_body(entity_hbm, rel_hbm, idxs_hbm, idxr_hbm, idxo_hbm,
                    out_s, out_r, out_o,
                    idxs_v, idxr_v, idxo_v, buf0, buf1,
                    gsem, wsem0, wsem1):
    wid = lax.axis_index("s") * _NC + lax.axis_index("c")
    base = wid * _BPW

    pltpu.sync_copy(idxs_hbm.at[pl.ds(base, _BPW)], idxs_v)
    pltpu.sync_copy(idxr_hbm.at[pl.ds(base, _BPW)], idxr_v)
    pltpu.sync_copy(idxo_hbm.at[pl.ds(base, _BPW)], idxo_v)

    jobs = (
        (entity_hbm, idxs_v, out_s),
        (rel_hbm, idxr_v, out_r),
        (entity_hbm, idxo_v, out_o),
    )
    bufs = (buf0, buf1)
    wsems = (wsem0, wsem1)
    pending = [None, None]
    step = 0
    for table, idx_v, out in jobs:
        for k in range(_NCHUNK):
            slot = step % 2
            if pending[slot] is not None:
                pending[slot].wait()
            buf = bufs[slot]
            pltpu.async_copy(
                table.at[idx_v.at[pl.ds(k * _CH, _CH)]], buf, gsem
            ).wait()
            pending[slot] = pltpu.async_copy(
                buf, out.at[pl.ds(base + k * _CH, _CH)], wsems[slot]
            )
            step += 1
    for p in pending:
        if p is not None:
            p.wait()


@functools.cache
def _sc_gather():
    return functools.partial(
        pl.kernel,
        out_type=[jax.ShapeDtypeStruct((_SB, EMBED_DIM), jnp.float32)] * 3,
        mesh=plsc.VectorSubcoreMesh(core_axis_name="c", subcore_axis_name="s",
                                    num_cores=_NC, num_subcores=_NS),
        scratch_types=[
            pltpu.VMEM((_BPW,), jnp.int32),
            pltpu.VMEM((_BPW,), jnp.int32),
            pltpu.VMEM((_BPW,), jnp.int32),
            pltpu.VMEM((_CH, EMBED_DIM), jnp.float32),
            pltpu.VMEM((_CH, EMBED_DIM), jnp.float32),
            pltpu.SemaphoreType.DMA,
            pltpu.SemaphoreType.DMA,
            pltpu.SemaphoreType.DMA,
        ],
    )(_sc_gather_body)


_BM = 512


def _mlp_body(hs_ref, hr_ref, ho_ref, w1s_ref, w1r_ref, w1o_ref,
              b1_ref, w2t_ref, b2_ref, out_ref):
    bf = jnp.bfloat16
    acc = jnp.dot(hs_ref[...].astype(bf), w1s_ref[...].astype(bf),
                  preferred_element_type=jnp.float32)
    acc += jnp.dot(hr_ref[...].astype(bf), w1r_ref[...].astype(bf),
                   preferred_element_type=jnp.float32)
    acc += jnp.dot(ho_ref[...].astype(bf), w1o_ref[...].astype(bf),
                   preferred_element_type=jnp.float32)
    hidden = jnp.maximum(acc + b1_ref[...], 0.0)
    out_ref[...] = jnp.sum(hidden * w2t_ref[...], axis=1) + b2_ref[0, 0]


def _mlp(hs, hr, ho, w1s, w1r, w1o, b1, w2t, b2):
    grid = (_SB // _BM,)
    return pl.pallas_call(
        _mlp_body,
        grid=grid,
        in_specs=[
            pl.BlockSpec((_BM, EMBED_DIM), lambda i: (i, 0)),
            pl.BlockSpec((_BM, EMBED_DIM), lambda i: (i, 0)),
            pl.BlockSpec((_BM, EMBED_DIM), lambda i: (i, 0)),
            pl.BlockSpec((EMBED_DIM, HIDDEN_DIM), lambda i: (0, 0)),
            pl.BlockSpec((EMBED_DIM, HIDDEN_DIM), lambda i: (0, 0)),
            pl.BlockSpec((EMBED_DIM, HIDDEN_DIM), lambda i: (0, 0)),
            pl.BlockSpec((1, HIDDEN_DIM), lambda i: (0, 0)),
            pl.BlockSpec((1, HIDDEN_DIM), lambda i: (0, 0)),
            pl.BlockSpec((1, 1), lambda i: (0, 0)),
        ],
        out_specs=pl.BlockSpec((_BM,), lambda i: (i,)),
        out_shape=jax.ShapeDtypeStruct((_SB,), jnp.float32),
    )(hs, hr, ho, w1s, w1r, w1o, b1, w2t, b2)


def kernel(entity_emb, triples, rel_emb, fc1, fc1_bias, fc2, fc2_bias):
    idx = triples.astype(jnp.int32)
    idx_s = idx[:, 0]
    idx_r = idx[:, 1]
    idx_o = idx[:, 2]
    w1s = fc1[:EMBED_DIM]
    w1r = fc1[EMBED_DIM:2 * EMBED_DIM]
    w1o = fc1[2 * EMBED_DIM:]
    b1 = fc1_bias.reshape(1, HIDDEN_DIM)
    w2t = fc2.reshape(1, HIDDEN_DIM)
    b2 = fc2_bias.reshape(1, 1)
    gather = _sc_gather()
    gathered = []
    for s in range(_SLABS):
        lo = s * _SB
        gathered.append(gather(entity_emb, rel_emb,
                               lax.slice(idx_s, (lo,), (lo + _SB,)),
                               lax.slice(idx_r, (lo,), (lo + _SB,)),
                               lax.slice(idx_o, (lo,), (lo + _SB,))))
    outs = [_mlp(hs, hr, ho, w1s, w1r, w1o, b1, w2t, b2)
            for hs, hr, ho in gathered]
    return jnp.concatenate(outs, axis=0)


# speedup vs baseline: 1.0974x; 1.0071x over previous
import functools

import jax
import jax.numpy as jnp
from jax import lax
from jax.experimental import pallas as pl
from jax.experimental.pallas import tpu as pltpu
from jax.experimental.pallas import tpu_sc as plsc

NUM_ENTITIES = 50000
EMBED_DIM = 512
HIDDEN_DIM = 1024
BATCH = 16384

_NC = 2
_NS = 16
_NW = _NC * _NS
_SLABS = 4
_SB = BATCH // _SLABS
_BPW = _SB // _NW
_CH = 64
_NCHUNK = _BPW // _CH


_NBUF = 2


def _sc_gather_body(entity_hbm, rel_hbm, idx_hbm,
                    out_s, out_r, out_o,
                    idx_v, buf0, buf1,
                    gsem0, gsem1, wsem0, wsem1):
    wid = lax.axis_index("s") * _NC + lax.axis_index("c")
    base = wid * _BPW

    pltpu.sync_copy(idx_hbm.at[pl.ds(wid * 3 * _BPW, 3 * _BPW)], idx_v)

    tables = (entity_hbm, rel_hbm, entity_hbm)
    outs = (out_s, out_r, out_o)
    bufs = (buf0, buf1)
    gsems = (gsem0, gsem1)
    wsems = (wsem0, wsem1)
    chunks = [(t, k) for t in range(3) for k in range(_NCHUNK)]
    total = len(chunks)

    pend_g = [None] * _NBUF
    pend_w = [None] * _NBUF
    for c in range(total + (_NBUF - 1)):
        if c < total:
            slot = c % _NBUF
            t, k = chunks[c]
            if pend_w[slot] is not None:
                pend_w[slot].wait()
            pend_g[slot] = pltpu.async_copy(
                tables[t].at[idx_v.at[pl.ds(t * _BPW + k * _CH, _CH)]],
                bufs[slot], gsems[slot])
        d = c - (_NBUF - 1)
        if 0 <= d < total:
            ds_ = d % _NBUF
            td, kd = chunks[d]
            pend_g[ds_].wait()
            pend_w[ds_] = pltpu.async_copy(
                bufs[ds_], outs[td].at[pl.ds(base + kd * _CH, _CH)],
                wsems[ds_])
    for p in pend_w:
        if p is not None:
            p.wait()


@functools.cache
def _sc_gather():
    return functools.partial(
        pl.kernel,
        out_type=[jax.ShapeDtypeStruct((_SB, EMBED_DIM), jnp.float32)] * 3,
        mesh=plsc.VectorSubcoreMesh(core_axis_name="c", subcore_axis_name="s",
                                    num_cores=_NC, num_subcores=_NS),
        scratch_types=[
            pltpu.VMEM((3 * _BPW,), jnp.int32),
            pltpu.VMEM((_CH, EMBED_DIM), jnp.float32),
            pltpu.VMEM((_CH, EMBED_DIM), jnp.float32),
            pltpu.SemaphoreType.DMA,
            pltpu.SemaphoreType.DMA,
            pltpu.SemaphoreType.DMA,
            pltpu.SemaphoreType.DMA,
        ],
    )(_sc_gather_body)


_BM = 512


def _mlp_body(hs_ref, hr_ref, ho_ref, w1s_ref, w1r_ref, w1o_ref,
              b1_ref, w2t_ref, b2_ref, out_ref):
    bf = jnp.bfloat16
    acc = jnp.dot(hs_ref[...].astype(bf), w1s_ref[...].astype(bf),
                  preferred_element_type=jnp.float32)
    acc += jnp.dot(hr_ref[...].astype(bf), w1r_ref[...].astype(bf),
                   preferred_element_type=jnp.float32)
    acc += jnp.dot(ho_ref[...].astype(bf), w1o_ref[...].astype(bf),
                   preferred_element_type=jnp.float32)
    hidden = jnp.maximum(acc + b1_ref[...], 0.0)
    out_ref[...] = jnp.sum(hidden * w2t_ref[...], axis=1) + b2_ref[0, 0]


def _mlp(hs, hr, ho, w1s, w1r, w1o, b1, w2t, b2):
    grid = (_SB // _BM,)
    return pl.pallas_call(
        _mlp_body,
        grid=grid,
        in_specs=[
            pl.BlockSpec((_BM, EMBED_DIM), lambda i: (i, 0)),
            pl.BlockSpec((_BM, EMBED_DIM), lambda i: (i, 0)),
            pl.BlockSpec((_BM, EMBED_DIM), lambda i: (i, 0)),
            pl.BlockSpec((EMBED_DIM, HIDDEN_DIM), lambda i: (0, 0)),
            pl.BlockSpec((EMBED_DIM, HIDDEN_DIM), lambda i: (0, 0)),
            pl.BlockSpec((EMBED_DIM, HIDDEN_DIM), lambda i: (0, 0)),
            pl.BlockSpec((1, HIDDEN_DIM), lambda i: (0, 0)),
            pl.BlockSpec((1, HIDDEN_DIM), lambda i: (0, 0)),
            pl.BlockSpec((1, 1), lambda i: (0, 0)),
        ],
        out_specs=pl.BlockSpec((_BM,), lambda i: (i,)),
        out_shape=jax.ShapeDtypeStruct((_SB,), jnp.float32),
    )(hs, hr, ho, w1s, w1r, w1o, b1, w2t, b2)


def kernel(entity_emb, triples, rel_emb, fc1, fc1_bias, fc2, fc2_bias):
    idx = triples.astype(jnp.int32)
    idx_packed = idx.reshape(_SLABS, _NW, _BPW, 3).transpose(0, 1, 3, 2)
    idx_packed = idx_packed.reshape(_SLABS, _NW * 3 * _BPW)
    w1s = fc1[:EMBED_DIM]
    w1r = fc1[EMBED_DIM:2 * EMBED_DIM]
    w1o = fc1[2 * EMBED_DIM:]
    b1 = fc1_bias.reshape(1, HIDDEN_DIM)
    w2t = fc2.reshape(1, HIDDEN_DIM)
    b2 = fc2_bias.reshape(1, 1)
    gather = _sc_gather()
    gathered = [gather(entity_emb, rel_emb, idx_packed[s])
                for s in range(_SLABS)]
    outs = [_mlp(hs, hr, ho, w1s, w1r, w1o, b1, w2t, b2)
            for hs, hr, ho in gathered]
    return jnp.concatenate(outs, axis=0)


# speedup vs baseline: 1.1013x; 1.0036x over previous
import functools

import jax
import jax.numpy as jnp
from jax import lax
from jax.experimental import pallas as pl
from jax.experimental.pallas import tpu as pltpu
from jax.experimental.pallas import tpu_sc as plsc

EMBED_DIM = 512
HIDDEN_DIM = 1024
BATCH = 16384

_NC = 2
_NS = 16
_NW = _NC * _NS
_CH = 64
_SLAB_SIZES = (6144, 4096, 4096, 2048)


def _make_gather_body(bpw, nchunk):
    def body(entity_hbm, rel_hbm, idx_hbm,
             out_s, out_r, out_o,
             idx_v, fbuf0, fbuf1,
             gsem0, gsem1, wsem0, wsem1):
        wid = lax.axis_index("s") * _NC + lax.axis_index("c")
        base = wid * bpw

        pltpu.sync_copy(idx_hbm.at[pl.ds(wid * 3 * bpw, 3 * bpw)], idx_v)

        tables = (entity_hbm, rel_hbm, entity_hbm)
        outs = (out_s, out_r, out_o)
        fbufs = (fbuf0, fbuf1)
        gsems = (gsem0, gsem1)
        wsems = (wsem0, wsem1)
        chunks = [(t, k) for t in range(3) for k in range(nchunk)]
        total = len(chunks)

        def issue_gather(c):
            t, k = chunks[c]
            return pltpu.async_copy(
                tables[t].at[idx_v.at[pl.ds(t * bpw + k * _CH, _CH)]],
                fbufs[c % 2], gsems[c % 2])

        pend_g = [None, None]
        pend_w = [None, None]
        pend_g[0] = issue_gather(0)
        for c in range(total):
            slot = c % 2
            if c + 1 < total:
                pend_g[(c + 1) % 2] = issue_gather(c + 1)
            pend_g[slot].wait()
            if pend_w[slot] is not None:
                pend_w[slot].wait()
            t, k = chunks[c]
            pend_w[slot] = pltpu.async_copy(
                fbufs[slot], outs[t].at[pl.ds(base + k * _CH, _CH)],
                wsems[slot])
        for p in pend_w:
            if p is not None:
                p.wait()

    return body


@functools.cache
def _sc_gather(sb):
    bpw = sb // _NW
    nchunk = bpw // _CH
    return functools.partial(
        pl.kernel,
        out_type=[jax.ShapeDtypeStruct((sb, EMBED_DIM), jnp.float32)] * 3,
        mesh=plsc.VectorSubcoreMesh(core_axis_name="c", subcore_axis_name="s",
                                    num_cores=_NC, num_subcores=_NS),
        scratch_types=[
            pltpu.VMEM((3 * bpw,), jnp.int32),
            pltpu.VMEM((_CH, EMBED_DIM), jnp.float32),
            pltpu.VMEM((_CH, EMBED_DIM), jnp.float32),
            pltpu.SemaphoreType.DMA,
            pltpu.SemaphoreType.DMA,
            pltpu.SemaphoreType.DMA,
            pltpu.SemaphoreType.DMA,
        ],
    )(_make_gather_body(bpw, nchunk))


_BM = 512


def _mlp_body(hs_ref, hr_ref, ho_ref, w1s_ref, w1r_ref, w1o_ref,
              b1_ref, w2t_ref, b2_ref, out_ref):
    bf = jnp.bfloat16
    acc = jnp.dot(hs_ref[...].astype(bf), w1s_ref[...].astype(bf),
                  preferred_element_type=jnp.float32)
    acc += jnp.dot(hr_ref[...].astype(bf), w1r_ref[...].astype(bf),
                   preferred_element_type=jnp.float32)
    acc += jnp.dot(ho_ref[...].astype(bf), w1o_ref[...].astype(bf),
                   preferred_element_type=jnp.float32)
    hidden = jnp.maximum(acc + b1_ref[...], 0.0)
    out_ref[...] = jnp.sum(hidden * w2t_ref[...], axis=1) + b2_ref[0, 0]


def _mlp(sb, hs, hr, ho, fc1, b1, w2t, b2):
    grid = (sb // _BM,)
    wspec = lambda t: pl.BlockSpec((EMBED_DIM, HIDDEN_DIM),
                                   lambda i, _t=t: (_t, 0))
    return pl.pallas_call(
        _mlp_body,
        grid=grid,
        in_specs=[
            pl.BlockSpec((_BM, EMBED_DIM), lambda i: (i, 0)),
            pl.BlockSpec((_BM, EMBED_DIM), lambda i: (i, 0)),
            pl.BlockSpec((_BM, EMBED_DIM), lambda i: (i, 0)),
            wspec(0),
            wspec(1),
            wspec(2),
            pl.BlockSpec((1, HIDDEN_DIM), lambda i: (0, 0)),
            pl.BlockSpec((1, HIDDEN_DIM), lambda i: (0, 0)),
            pl.BlockSpec((1, 1), lambda i: (0, 0)),
        ],
        out_specs=pl.BlockSpec((_BM,), lambda i: (i,)),
        out_shape=jax.ShapeDtypeStruct((sb,), jnp.float32),
    )(hs, hr, ho, fc1, fc1, fc1, b1, w2t, b2)


def kernel(entity_emb, triples, rel_emb, fc1, fc1_bias, fc2, fc2_bias):
    idx = triples.astype(jnp.int32)
    b1 = fc1_bias.reshape(1, HIDDEN_DIM)
    w2t = fc2.reshape(1, HIDDEN_DIM)
    b2 = fc2_bias.reshape(1, 1)

    gathered = []
    lo = 0
    for sb in _SLAB_SIZES:
        bpw = sb // _NW
        idx_slab = lax.slice(idx, (lo, 0), (lo + sb, 3))
        idx_packed = idx_slab.reshape(_NW, bpw, 3).transpose(0, 2, 1)
        idx_packed = idx_packed.reshape(_NW * 3 * bpw)
        gathered.append(_sc_gather(sb)(entity_emb, rel_emb, idx_packed))
        lo += sb
    outs = [_mlp(sb, hs, hr, ho, fc1, b1, w2t, b2)
            for sb, (hs, hr, ho) in zip(_SLAB_SIZES, gathered)]
    return jnp.concatenate(outs, axis=0)


# speedup vs baseline: 1.1202x; 1.0172x over previous
import functools

import jax
import jax.numpy as jnp
from jax import lax
from jax.experimental import pallas as pl
from jax.experimental.pallas import tpu as pltpu
from jax.experimental.pallas import tpu_sc as plsc

EMBED_DIM = 512
HIDDEN_DIM = 1024
BATCH = 16384

_NC = 2
_NS = 16
_NW = _NC * _NS
_CH = 64
_SLAB_SIZES = (6144, 4096, 4096, 2048)


def _make_gather_body(bpw, nchunk):
    def body(entity_hbm, rel_hbm, idx_hbm,
             out_s, out_r, out_o,
             idx_v, fbuf0, fbuf1, fbuf2,
             gsem0, gsem1, gsem2, wsem0, wsem1, wsem2):
        wid = lax.axis_index("s") * _NC + lax.axis_index("c")
        base = wid * bpw

        pltpu.sync_copy(idx_hbm.at[pl.ds(wid * 3 * bpw, 3 * bpw)], idx_v)

        tables = (entity_hbm, rel_hbm, entity_hbm)
        outs = (out_s, out_r, out_o)
        fbufs = (fbuf0, fbuf1, fbuf2)
        gsems = (gsem0, gsem1, gsem2)
        wsems = (wsem0, wsem1, wsem2)
        nbuf = 3
        chunks = [(t, k) for t in range(3) for k in range(nchunk)]
        total = len(chunks)

        pend_g = [None] * nbuf
        pend_w = [None] * nbuf
        for c in range(total + nbuf - 1):
            if c < total:
                slot = c % nbuf
                if pend_w[slot] is not None:
                    pend_w[slot].wait()
                t, k = chunks[c]
                pend_g[slot] = pltpu.async_copy(
                    tables[t].at[idx_v.at[pl.ds(t * bpw + k * _CH, _CH)]],
                    fbufs[slot], gsems[slot])
            d = c - (nbuf - 1)
            if 0 <= d < total:
                ds_ = d % nbuf
                td, kd = chunks[d]
                pend_g[ds_].wait()
                pend_w[ds_] = pltpu.async_copy(
                    fbufs[ds_], outs[td].at[pl.ds(base + kd * _CH, _CH)],
                    wsems[ds_])
        for p in pend_w:
            if p is not None:
                p.wait()

    return body


@functools.cache
def _sc_gather(sb):
    bpw = sb // _NW
    nchunk = bpw // _CH
    return functools.partial(
        pl.kernel,
        out_type=[jax.ShapeDtypeStruct((sb, EMBED_DIM), jnp.float32)] * 3,
        mesh=plsc.VectorSubcoreMesh(core_axis_name="c", subcore_axis_name="s",
                                    num_cores=_NC, num_subcores=_NS),
        scratch_types=[
            pltpu.VMEM((3 * bpw,), jnp.int32),
            pltpu.VMEM((_CH, EMBED_DIM), jnp.float32),
            pltpu.VMEM((_CH, EMBED_DIM), jnp.float32),
            pltpu.VMEM((_CH, EMBED_DIM), jnp.float32),
            pltpu.SemaphoreType.DMA,
            pltpu.SemaphoreType.DMA,
            pltpu.SemaphoreType.DMA,
            pltpu.SemaphoreType.DMA,
            pltpu.SemaphoreType.DMA,
            pltpu.SemaphoreType.DMA,
        ],
    )(_make_gather_body(bpw, nchunk))


_BM = 512


def _mlp_body(hs_ref, hr_ref, ho_ref, w1s_ref, w1r_ref, w1o_ref,
              b1_ref, w2t_ref, b2_ref, out_ref):
    bf = jnp.bfloat16
    acc = jnp.dot(hs_ref[...].astype(bf), w1s_ref[...].astype(bf),
                  preferred_element_type=jnp.float32)
    acc += jnp.dot(hr_ref[...].astype(bf), w1r_ref[...].astype(bf),
                   preferred_element_type=jnp.float32)
    acc += jnp.dot(ho_ref[...].astype(bf), w1o_ref[...].astype(bf),
                   preferred_element_type=jnp.float32)
    hidden = jnp.maximum(acc + b1_ref[...], 0.0)
    out_ref[...] = jnp.sum(hidden * w2t_ref[...], axis=1) + b2_ref[0, 0]


def _mlp(sb, hs, hr, ho, fc1, b1, w2t, b2):
    grid = (sb // _BM,)
    wspec = lambda t: pl.BlockSpec((EMBED_DIM, HIDDEN_DIM),
                                   lambda i, _t=t: (_t, 0))
    return pl.pallas_call(
        _mlp_body,
        grid=grid,
        in_specs=[
            pl.BlockSpec((_BM, EMBED_DIM), lambda i: (i, 0)),
            pl.BlockSpec((_BM, EMBED_DIM), lambda i: (i, 0)),
            pl.BlockSpec((_BM, EMBED_DIM), lambda i: (i, 0)),
            wspec(0),
            wspec(1),
            wspec(2),
            pl.BlockSpec((1, HIDDEN_DIM), lambda i: (0, 0)),
            pl.BlockSpec((1, HIDDEN_DIM), lambda i: (0, 0)),
            pl.BlockSpec((1, 1), lambda i: (0, 0)),
        ],
        out_specs=pl.BlockSpec((_BM,), lambda i: (i,)),
        out_shape=jax.ShapeDtypeStruct((sb,), jnp.float32),
    )(hs, hr, ho, fc1, fc1, fc1, b1, w2t, b2)


def kernel(entity_emb, triples, rel_emb, fc1, fc1_bias, fc2, fc2_bias):
    idx = triples.astype(jnp.int32)
    b1 = fc1_bias.reshape(1, HIDDEN_DIM)
    w2t = fc2.reshape(1, HIDDEN_DIM)
    b2 = fc2_bias.reshape(1, 1)

    gathered = []
    lo = 0
    for sb in _SLAB_SIZES:
        bpw = sb // _NW
        idx_slab = lax.slice(idx, (lo, 0), (lo + sb, 3))
        idx_packed = idx_slab.reshape(_NW, bpw, 3).transpose(0, 2, 1)
        idx_packed = idx_packed.reshape(_NW * 3 * bpw)
        gathered.append(_sc_gather(sb)(entity_emb, rel_emb, idx_packed))
        lo += sb
    outs = [_mlp(sb, hs, hr, ho, fc1, b1, w2t, b2)
            for sb, (hs, hr, ho) in zip(_SLAB_SIZES, gathered)]
    return jnp.concatenate(outs, axis=0)
